# baseline (speedup 1.0000x reference)
import functools

import jax
import jax.numpy as jnp
from jax import lax
from jax.experimental import pallas as pl
from jax.experimental.pallas import tpu as pltpu
from jax.experimental.pallas import tpu_sc as plsc

_NW = 32
_CH = 32
_NBUF = 2


def _make_sc_main(b, d, sw):
    nch = sw // _CH
    mesh = plsc.VectorSubcoreMesh(core_axis_name="c", subcore_axis_name="s")

    @functools.partial(
        pl.kernel,
        mesh=mesh,
        out_type=jax.ShapeDtypeStruct((b, 64), jnp.float32),
        scratch_types=[
            pltpu.VMEM((7, sw), jnp.int32),
            pltpu.VMEM((_NBUF, 7, _CH, d), jnp.float32),
            pltpu.VMEM((2, _CH, 64), jnp.float32),
            pltpu.SemaphoreType.DMA,
            pltpu.SemaphoreType.DMA,
            pltpu.SemaphoreType.DMA,
            pltpu.SemaphoreType.DMA,
        ],
    )
    def sc_main(table, idx_hbm, pb_out,
                idx_v, rows_v, pbuf, sem0, sem1, semw0, semw1):
        info = plsc.get_sparse_core_info()
        nc = info.num_cores
        wid = lax.axis_index("s") * nc + lax.axis_index("c")
        base = pl.multiple_of(wid * sw, sw)
        sems = (sem0, sem1)
        semws = (semw0, semw1)

        pltpu.sync_copy(idx_hbm.at[:, pl.ds(base, sw)], idx_v)

        def issue(c, slot):
            off = pl.multiple_of(c * _CH, _CH)
            for q in range(7):
                pltpu.async_copy(
                    table.at[idx_v.at[q, pl.ds(off, _CH)]],
                    rows_v.at[slot, q],
                    sems[slot],
                )

        def wait_slot(slot):
            for q in range(7):
                pltpu.make_async_copy(
                    table.at[idx_v.at[q, pl.ds(0, _CH)]],
                    rows_v.at[slot, q],
                    sems[slot],
                ).wait()

        def issue_write(c, slot):
            off = pl.multiple_of(base + c * _CH, _CH)
            pltpu.async_copy(pbuf.at[slot], pb_out.at[pl.ds(off, _CH)],
                             semws[slot])

        def wait_write(slot):
            pltpu.make_async_copy(pbuf.at[slot],
                                  pb_out.at[pl.ds(0, _CH)],
                                  semws[slot]).wait()

        zero16 = jnp.zeros((16,), jnp.float32)

        def compute_chunk(c, slot, wslot):
            wait_slot(slot)

            @pl.when(c >= 2)
            def _():
                wait_write(wslot)

            def sample_body(s, _):
                acc_p = zero16
                acc_n = zero16
                acc_1 = zero16
                acc_2 = zero16
                for k in range(8):
                    sl_ = pl.ds(k * 16, 16)
                    ra = rows_v[slot, 0, s, sl_]
                    rb = rows_v[slot, 1, s, sl_]
                    rc = rows_v[slot, 2, s, sl_]
                    rd = rows_v[slot, 3, s, sl_]
                    ri = rows_v[slot, 4, s, sl_]
                    rj = rows_v[slot, 5, s, sl_]
                    rl = rows_v[slot, 6, s, sl_]
                    dp = ra - rb
                    dn = rc - rd
                    acc_p = acc_p + dp * dp
                    acc_n = acc_n + dn * dn
                    acc_1 = acc_1 + (ri - rj)
                    acc_2 = acc_2 + (ri - rl)
                pbuf[wslot, s, pl.ds(0, 16)] = acc_p
                pbuf[wslot, s, pl.ds(16, 16)] = acc_n
                pbuf[wslot, s, pl.ds(32, 16)] = acc_1
                pbuf[wslot, s, pl.ds(48, 16)] = acc_2
                return 0

            lax.fori_loop(0, _CH, sample_body, 0, unroll=False)
            issue_write(c, wslot)

        for s0 in range(_NBUF):
            issue(s0, s0)

        def steady(c4, _):
            for sl in range(_NBUF):
                c = c4 * _NBUF + sl
                compute_chunk(c, sl, sl % 2)

                @pl.when(c4 < nch // _NBUF - 1)
                def _():
                    issue(c + _NBUF, sl)

            return 0

        lax.fori_loop(0, nch // _NBUF, steady, 0, unroll=False)
        wait_write(0)
        wait_write(1)

    return sc_main


def _lsmooth_body(e_ref, o_ref):
    i = pl.program_id(0)
    d = e_ref[1:, :, :] - e_ref[:-1, :, :]
    s = jnp.sum(d * d)

    @pl.when(i == 0)
    def _():
        o_ref[:, :] = jnp.zeros((1, 1), jnp.float32)

    o_ref[:, :] += jnp.reshape(s, (1, 1))


def _combine_body(b, d, t, v, pb_ref, w_ref, c0_ref, c1_ref, c2_ref,
                  th_ref, be_ref, lsm_ref, o_ref):
    i = pl.program_id(0)
    pbb = pb_ref[0]
    row = lax.broadcasted_iota(jnp.int32, (4, 64), 0)
    col = lax.broadcasted_iota(jnp.int32, (4, 64), 1)
    mt = jnp.where(col // 16 == row, 1.0, 0.0).astype(jnp.float32)
    fold = lax.dot_general(mt, pbb, (((1,), (1,)), ((), ())),
                           preferred_element_type=jnp.float32)
    dist_p = fold[0]
    dist_n = fold[1]
    t1 = fold[2]
    t2 = fold[3]

    m = jnp.maximum(dist_p - dist_n + 1.0, 0.0)
    lprox_part = jnp.sum(m * w_ref[0, 0])

    s = c1_ref[0, 0] * t1 + c2_ref[0, 0] * t2
    ip = th_ref[:, :] * s[None, :] + be_ref[0, 0]
    ip = jnp.clip(ip, -50.0, 50.0)
    lp = jnp.log(1.0 + jnp.exp(-ip.astype(jnp.bfloat16)))
    lt_part = (jnp.sum(c0_ref[0, 0][None, :] * ip)
               + jnp.sum(lp.astype(jnp.float32)))

    @pl.when(i == 0)
    def _():
        o_ref[:, :] = lsm_ref[:, :] / ((t - 1) * v)

    o_ref[:, :] += jnp.reshape(lprox_part / b + lt_part / (b * d), (1, 1))


def kernel(data, weight, triag_int, triag_float, embedding, theta, beta):
    t, v, d = embedding.shape
    b = data.shape[0]
    sw = b // _NW

    data = data.astype(jnp.int32)
    triag_int = triag_int.astype(jnp.int32)

    ia = data[:, 0] * v + data[:, 1]
    ib = data[:, 0] * v + data[:, 2]
    ic = data[:, 0] * v + data[:, 3]
    idd = data[:, 0] * v + data[:, 4]
    ti = triag_int[:, 0] * v + triag_int[:, 1]
    tj = triag_int[:, 0] * v + triag_int[:, 2]
    tl = triag_int[:, 0] * v + triag_int[:, 3]
    idxs = jnp.stack([ia, ib, ic, idd, ti, tj, tl])

    table = embedding.reshape(t * v, d)
    pb = _make_sc_main(b, d, sw)(table, idxs)

    vb = 2000
    lsm = pl.pallas_call(
        _lsmooth_body,
        grid=(v // vb,),
        in_specs=[pl.BlockSpec((t, vb, d), lambda i: (0, i, 0))],
        out_specs=pl.BlockSpec((1, 1), lambda i: (0, 0)),
        out_shape=jax.ShapeDtypeStruct((1, 1), jnp.float32),
    )(embedding)

    bb = 8192
    nb = b // bb
    pb3 = pb.reshape(nb, bb, 64)
    w3 = weight.reshape(nb, 1, bb)
    c0 = triag_float[:, 0].reshape(nb, 1, bb)
    c1 = triag_float[:, 1].reshape(nb, 1, bb)
    c2 = triag_float[:, 2].reshape(nb, 1, bb)
    th2 = theta.reshape(d, 1)
    be2 = beta.reshape(1, 1)

    col_spec = pl.BlockSpec((1, 1, bb), lambda i: (i, 0, 0))
    loss = pl.pallas_call(
        functools.partial(_combine_body, b, d, t, v),
        grid=(nb,),
        in_specs=[
            pl.BlockSpec((1, bb, 64), lambda i: (i, 0, 0)),
            col_spec,
            col_spec,
            col_spec,
            col_spec,
            pl.BlockSpec((d, 1), lambda i: (0, 0)),
            pl.BlockSpec((1, 1), lambda i: (0, 0)),
            pl.BlockSpec((1, 1), lambda i: (0, 0)),
        ],
        out_specs=pl.BlockSpec((1, 1), lambda i: (0, 0)),
        out_shape=jax.ShapeDtypeStruct((1, 1), jnp.float32),
    )(pb3, w3, c0, c1, c2, th2, be2, lsm)

    return jnp.reshape(loss, ())


# speedup vs baseline: 1.0218x; 1.0218x over previous
import functools

import jax
import jax.numpy as jnp
from jax import lax
from jax.experimental import pallas as pl
from jax.experimental.pallas import tpu as pltpu
from jax.experimental.pallas import tpu_sc as plsc

_NW = 32
_CH = 32
_NBUF = 2


def _make_sc_main(b, d, sw):
    nch = sw // _CH
    mesh = plsc.VectorSubcoreMesh(core_axis_name="c", subcore_axis_name="s")

    @functools.partial(
        pl.kernel,
        mesh=mesh,
        out_type=jax.ShapeDtypeStruct((b, 64), jnp.float32),
        scratch_types=[
            pltpu.VMEM((7, sw), jnp.int32),
            pltpu.VMEM((_NBUF, 7, _CH, d), jnp.float32),
            pltpu.VMEM((2, _CH, 64), jnp.float32),
            pltpu.SemaphoreType.DMA,
            pltpu.SemaphoreType.DMA,
            pltpu.SemaphoreType.DMA,
            pltpu.SemaphoreType.DMA,
        ],
    )
    def sc_main(table, idx_hbm, pb_out,
                idx_v, rows_v, pbuf, sem0, sem1, semw0, semw1):
        info = plsc.get_sparse_core_info()
        nc = info.num_cores
        wid = lax.axis_index("s") * nc + lax.axis_index("c")
        base = pl.multiple_of(wid * sw, sw)
        sems = (sem0, sem1)
        semws = (semw0, semw1)

        pltpu.sync_copy(idx_hbm.at[:, pl.ds(base, sw)], idx_v)

        def issue(c, slot):
            off = pl.multiple_of(c * _CH, _CH)
            for q in range(7):
                pltpu.async_copy(
                    table.at[idx_v.at[q, pl.ds(off, _CH)]],
                    rows_v.at[slot, q],
                    sems[slot],
                )

        def wait_slot(slot):
            for q in range(7):
                pltpu.make_async_copy(
                    table.at[idx_v.at[q, pl.ds(0, _CH)]],
                    rows_v.at[slot, q],
                    sems[slot],
                ).wait()

        def issue_write(c, slot):
            off = pl.multiple_of(base + c * _CH, _CH)
            pltpu.async_copy(pbuf.at[slot], pb_out.at[pl.ds(off, _CH)],
                             semws[slot])

        def wait_write(slot):
            pltpu.make_async_copy(pbuf.at[slot],
                                  pb_out.at[pl.ds(0, _CH)],
                                  semws[slot]).wait()

        zero16 = jnp.zeros((16,), jnp.float32)

        def compute_chunk(c, slot, wslot):
            wait_slot(slot)

            @pl.when(c >= 2)
            def _():
                wait_write(wslot)

            def sample_body(s, _):
                acc_p = zero16
                acc_n = zero16
                acc_1 = zero16
                acc_2 = zero16
                for k in range(8):
                    sl_ = pl.ds(k * 16, 16)
                    ra = rows_v[slot, 0, s, sl_]
                    rb = rows_v[slot, 1, s, sl_]
                    rc = rows_v[slot, 2, s, sl_]
                    rd = rows_v[slot, 3, s, sl_]
                    ri = rows_v[slot, 4, s, sl_]
                    rj = rows_v[slot, 5, s, sl_]
                    rl = rows_v[slot, 6, s, sl_]
                    dp = ra - rb
                    dn = rc - rd
                    acc_p = acc_p + dp * dp
                    acc_n = acc_n + dn * dn
                    acc_1 = acc_1 + (ri - rj)
                    acc_2 = acc_2 + (ri - rl)
                pbuf[wslot, s, pl.ds(0, 16)] = acc_p
                pbuf[wslot, s, pl.ds(16, 16)] = acc_n
                pbuf[wslot, s, pl.ds(32, 16)] = acc_1
                pbuf[wslot, s, pl.ds(48, 16)] = acc_2
                return 0

            lax.fori_loop(0, _CH, sample_body, 0, unroll=False)
            issue_write(c, wslot)

        for s0 in range(_NBUF):
            issue(s0, s0)

        def steady(c4, _):
            for sl in range(_NBUF):
                c = c4 * _NBUF + sl
                compute_chunk(c, sl, sl % 2)

                @pl.when(c4 < nch // _NBUF - 1)
                def _():
                    issue(c + _NBUF, sl)

            return 0

        lax.fori_loop(0, nch // _NBUF, steady, 0, unroll=False)
        wait_write(0)
        wait_write(1)

    return sc_main


def _lsmooth_body(e_ref, o_ref):
    i = pl.program_id(0)
    d = e_ref[1:, :, :] - e_ref[:-1, :, :]
    s = jnp.sum(d * d)

    @pl.when(i == 0)
    def _():
        o_ref[:, :] = jnp.zeros((1, 1), jnp.float32)

    o_ref[:, :] += jnp.reshape(s, (1, 1))


def _combine_body(b, d, t, v, pb_ref, w_ref, c0_ref, c1_ref, c2_ref,
                  th_ref, be_ref, lsm_ref, o_ref):
    i = pl.program_id(0)
    pbb = pb_ref[0]
    row = lax.broadcasted_iota(jnp.int32, (4, 64), 0)
    col = lax.broadcasted_iota(jnp.int32, (4, 64), 1)
    mt = jnp.where(col // 16 == row, 1.0, 0.0).astype(jnp.float32)
    fold = lax.dot_general(mt, pbb, (((1,), (1,)), ((), ())),
                           preferred_element_type=jnp.float32)
    dist_p = fold[0]
    dist_n = fold[1]
    t1 = fold[2]
    t2 = fold[3]

    m = jnp.maximum(dist_p - dist_n + 1.0, 0.0)
    lprox_part = jnp.sum(m * w_ref[0, 0])

    s = c1_ref[0, 0] * t1 + c2_ref[0, 0] * t2
    ip = th_ref[:, :] * s[None, :] + be_ref[0, 0]
    ip = jnp.clip(ip, -50.0, 50.0)
    lp = jnp.log(1.0 + jnp.exp(-ip))
    lt_part = jnp.sum(c0_ref[0, 0][None, :] * ip + lp)

    @pl.when(i == 0)
    def _():
        o_ref[:, :] = lsm_ref[:, :] / ((t - 1) * v)

    o_ref[:, :] += jnp.reshape(lprox_part / b + lt_part / (b * d), (1, 1))


def kernel(data, weight, triag_int, triag_float, embedding, theta, beta):
    t, v, d = embedding.shape
    b = data.shape[0]
    sw = b // _NW

    data = data.astype(jnp.int32)
    triag_int = triag_int.astype(jnp.int32)

    ia = data[:, 0] * v + data[:, 1]
    ib = data[:, 0] * v + data[:, 2]
    ic = data[:, 0] * v + data[:, 3]
    idd = data[:, 0] * v + data[:, 4]
    ti = triag_int[:, 0] * v + triag_int[:, 1]
    tj = triag_int[:, 0] * v + triag_int[:, 2]
    tl = triag_int[:, 0] * v + triag_int[:, 3]
    idxs = jnp.stack([ia, ib, ic, idd, ti, tj, tl])

    table = embedding.reshape(t * v, d)
    pb = _make_sc_main(b, d, sw)(table, idxs)

    vb = 2000
    lsm = pl.pallas_call(
        _lsmooth_body,
        grid=(v // vb,),
        in_specs=[pl.BlockSpec((t, vb, d), lambda i: (0, i, 0))],
        out_specs=pl.BlockSpec((1, 1), lambda i: (0, 0)),
        out_shape=jax.ShapeDtypeStruct((1, 1), jnp.float32),
    )(embedding)

    bb = 8192
    nb = b // bb
    pb3 = pb.reshape(nb, bb, 64)
    w3 = weight.reshape(nb, 1, bb)
    c0 = triag_float[:, 0].reshape(nb, 1, bb)
    c1 = triag_float[:, 1].reshape(nb, 1, bb)
    c2 = triag_float[:, 2].reshape(nb, 1, bb)
    th2 = theta.reshape(d, 1)
    be2 = beta.reshape(1, 1)

    col_spec = pl.BlockSpec((1, 1, bb), lambda i: (i, 0, 0))
    loss = pl.pallas_call(
        functools.partial(_combine_body, b, d, t, v),
        grid=(nb,),
        in_specs=[
            pl.BlockSpec((1, bb, 64), lambda i: (i, 0, 0)),
            col_spec,
            col_spec,
            col_spec,
            col_spec,
            pl.BlockSpec((d, 1), lambda i: (0, 0)),
            pl.BlockSpec((1, 1), lambda i: (0, 0)),
            pl.BlockSpec((1, 1), lambda i: (0, 0)),
        ],
        out_specs=pl.BlockSpec((1, 1), lambda i: (0, 0)),
        out_shape=jax.ShapeDtypeStruct((1, 1), jnp.float32),
    )(pb3, w3, c0, c1, c2, th2, be2, lsm)

    return jnp.reshape(loss, ())
